# baseline (speedup 1.0000x reference)
import functools

import jax
import jax.numpy as jnp
from jax import lax
from jax.experimental import pallas as pl
from jax.experimental.pallas import tpu as pltpu
from jax.experimental.pallas import tpu_sc as plsc

T = 4
N_T = 1024
E = 65536
L = 2
H = 16

_N_TILES = 32
_T_SPLIT = 2
_TILES_PER_T = _N_TILES // _T_SPLIT
_E_PER_TILE = E // _TILES_PER_T
_GROUPS = _E_PER_TILE // 16
_ACC = 10 * N_T


def _sc_edge_body(pair, x0_hbm, x1_hbm, src_hbm, dst_hbm, ea0_hbm, ea1_hbm,
                  wq_hbm, wk_hbm, we_hbm, bq_hbm, bk_hbm, out_hbm, x0_v, x1_v,
                  wq_v, wk_v, we_v, bq_v, bk_v, acc_v, src_v, dst_v, ea0_v,
                  ea1_v):
    cid = lax.axis_index("c")
    sid = lax.axis_index("s")
    wid = sid * 2 + cid
    t = wid // _TILES_PER_T
    tg = pair * 2 + t
    s = wid % _TILES_PER_T

    pltpu.sync_copy(x0_hbm, x0_v)
    pltpu.sync_copy(x1_hbm, x1_v)
    pltpu.sync_copy(wq_hbm, wq_v)
    pltpu.sync_copy(wk_hbm, wk_v)
    pltpu.sync_copy(we_hbm, we_v)
    pltpu.sync_copy(bq_hbm, bq_v)
    pltpu.sync_copy(bk_hbm, bk_v)
    pltpu.sync_copy(src_hbm.at[t, pl.ds(s * _E_PER_TILE, _E_PER_TILE)], src_v)
    pltpu.sync_copy(dst_hbm.at[t, pl.ds(s * _E_PER_TILE, _E_PER_TILE)], dst_v)
    pltpu.sync_copy(ea0_hbm.at[t, pl.ds(s * _E_PER_TILE, _E_PER_TILE)], ea0_v)
    pltpu.sync_copy(ea1_hbm.at[t, pl.ds(s * _E_PER_TILE, _E_PER_TILE)], ea1_v)

    scale = 0.25

    def dot16(a, b):
        return jnp.sum(a * b) * scale

    cof = []
    for l in range(L):
        wq0 = wq_v[pl.ds(l * 128 + tg * 32, 16)]
        wq1 = wq_v[pl.ds(l * 128 + tg * 32 + 16, 16)]
        bq = bq_v[pl.ds(l * 64 + tg * 16, 16)]
        bk = bk_v[pl.ds(l * 64 + tg * 16, 16)]
        slots = []
        for (wsrc, rows) in ((wk_v, 2), (we_v, 2)):
            for e in range(rows):
                we_row = wsrc[pl.ds(l * 128 + tg * 32 + e * 16, 16)]
                slots.append((dot16(wq0, we_row), dot16(wq1, we_row),
                              dot16(bq, we_row)))
        slots.append((dot16(wq0, bk), dot16(wq1, bk), dot16(bq, bk)))
        cof.append(slots)

    xbase = tg * N_T
    zero = jnp.zeros((16,), jnp.float32)

    def z_body(i, carry):
        acc_v[pl.ds(i * 16, 16)] = zero
        return carry

    lax.fori_loop(0, _ACC // 16, z_body, 0)

    def one_group(b):
        srci = src_v[pl.ds(b, 16)]
        dsti = dst_v[pl.ds(b, 16)]
        xs0 = plsc.load_gather(x0_v, [srci])
        xs1 = plsc.load_gather(x1_v, [srci])
        di = xbase + dsti
        xd0 = plsc.load_gather(x0_v, [di])
        xd1 = plsc.load_gather(x1_v, [di])
        ea0 = ea0_v[pl.ds(b, 16)]
        ea1 = ea1_v[pl.ds(b, 16)]
        exs = []
        for l in range(L):
            u0, u1, w0, w1, s0 = [xd0 * m0 + (xd1 * m1 + c)
                                  for (m0, m1, c) in cof[l]]
            exs.append(jnp.exp((u0 * xs0 + u1 * xs1)
                               + ((w0 * ea0 + w1 * ea1) + s0)))
        for l in range(L):
            ex = exs[l]
            o = l * 5 * N_T
            plsc.addupdate_scatter(acc_v.at[pl.ds(o, N_T)], [dsti], ex)
            plsc.addupdate_scatter(acc_v.at[pl.ds(o + N_T, N_T)], [dsti],
                                   ex * xs0)
            plsc.addupdate_scatter(acc_v.at[pl.ds(o + 2 * N_T, N_T)], [dsti],
                                   ex * xs1)
            plsc.addupdate_scatter(acc_v.at[pl.ds(o + 3 * N_T, N_T)], [dsti],
                                   ex * ea0)
            plsc.addupdate_scatter(acc_v.at[pl.ds(o + 4 * N_T, N_T)], [dsti],
                                   ex * ea1)

    _UNROLL = 4

    def e_body(g, carry):
        b = g * (16 * _UNROLL)
        for u in range(_UNROLL):
            one_group(b + u * 16)
        return carry

    lax.fori_loop(0, _GROUPS // _UNROLL, e_body, 0)

    pltpu.sync_copy(acc_v, out_hbm.at[wid])


def _make_sc_edge(pair):
  @jax.jit
  def _sc_edge(x0, x1, src, dst, ea0, ea1, wq, wk, we, bq, bk):
    mesh = plsc.VectorSubcoreMesh(core_axis_name="c", subcore_axis_name="s")
    return pl.kernel(
        functools.partial(_sc_edge_body, pair),
        out_type=jax.ShapeDtypeStruct((_N_TILES, _ACC), jnp.float32),
        mesh=mesh,
        scratch_types=[
            pltpu.VMEM((T * N_T,), jnp.float32),
            pltpu.VMEM((T * N_T,), jnp.float32),
            pltpu.VMEM((L * T * 32,), jnp.float32),
            pltpu.VMEM((L * T * 32,), jnp.float32),
            pltpu.VMEM((L * T * 32,), jnp.float32),
            pltpu.VMEM((L * T * 16,), jnp.float32),
            pltpu.VMEM((L * T * 16,), jnp.float32),
            pltpu.VMEM((_ACC,), jnp.float32),
            pltpu.VMEM((_E_PER_TILE,), jnp.int32),
            pltpu.VMEM((_E_PER_TILE,), jnp.int32),
            pltpu.VMEM((_E_PER_TILE,), jnp.float32),
            pltpu.VMEM((_E_PER_TILE,), jnp.float32),
        ],
        compiler_params=pltpu.CompilerParams(needs_layout_passes=False),
    )(x0, x1, src, dst, ea0, ea1, wq, wk, we, bq, bk)
  return _sc_edge


_sc_edge_p = (_make_sc_edge(0), _make_sc_edge(1))


def _fin_body(part_ref, x0_ref, x1_ref, wv_ref, we_ref, ws_ref, bv_ref,
              bs_ref, out_ref):
    part = part_ref[...]
    nt = x0_ref.shape[0] // N_T
    tiles = part.shape[0] // nt
    xd0 = x0_ref[...].reshape(nt, N_T)
    xd1 = x1_ref[...].reshape(nt, N_T)

    def tot(k):
        sl = part[:, k * N_T:(k + 1) * N_T].reshape(nt, tiles, N_T)
        return jnp.sum(sl, axis=1)

    for l in range(L):
        o = l * 5
        den = tot(o)
        r = 1.0 / (den + 1e-16)
        mh0 = tot(o + 1) * r
        mh1 = tot(o + 2) * r
        nh0 = tot(o + 3) * r
        nh1 = tot(o + 4) * r
        sa = den * r
        for pq in range(2):
            lo, hi = pq * (H // 2), pq * (H // 2) + H // 2

            def half(ref, j):
                return jnp.sum(ref[l, :, j, lo:hi], axis=-1)[:, None]

            def halfb(ref):
                return jnp.sum(ref[l, :, lo:hi], axis=-1)[:, None]

            val = (mh0 * half(wv_ref, 0) + mh1 * half(wv_ref, 1)
                   + nh0 * half(we_ref, 0) + nh1 * half(we_ref, 1)
                   + sa * halfb(bv_ref)
                   + xd0 * half(ws_ref, 0) + xd1 * half(ws_ref, 1)
                   + halfb(bs_ref))
            out_ref[:, l, pq] = val


@jax.jit
def _finalize(part, x0, x1, Wv, We, Ws, bv, bs):
    return pl.pallas_call(
        _fin_body,
        out_shape=jax.ShapeDtypeStruct((x0.shape[0] // N_T, L, 2, N_T),
                                       jnp.float32),
    )(part, x0, x1, Wv, We, Ws, bv, bs)


def _mv_body(f_ref, w_ref, b_ref, out_ref):
    out_ref[0] = (
        jnp.dot(f_ref[0], w_ref[0], preferred_element_type=jnp.float32)
        + b_ref[0]
    )


@jax.jit
def _matvec(flat, fcW, fcb):
    k = L * N_T * 2
    n = N_T * 2
    cb = 512
    nt = flat.shape[0]
    out = pl.pallas_call(
        _mv_body,
        grid=(nt, n // cb),
        in_specs=[
            pl.BlockSpec((1, 1, k), lambda t, c: (t, 0, 0)),
            pl.BlockSpec((1, k, cb), lambda t, c: (t, 0, c)),
            pl.BlockSpec((1, 1, cb), lambda t, c: (t, 0, c)),
        ],
        out_specs=pl.BlockSpec((1, 1, cb), lambda t, c: (t, 0, c)),
        out_shape=jax.ShapeDtypeStruct((nt, 1, n), jnp.float32),
    )(flat.reshape(nt, 1, k), fcW, fcb.reshape(nt, 1, n))
    return out.reshape(nt, n)


def kernel(x, edge_src, edge_dst, edge_attr, Wq, Wk, Wv, We, Ws,
           bq, bk, bv, bs, fcW, fcb):
    x0 = x[:, 0].astype(jnp.float32)
    x1 = x[:, 1].astype(jnp.float32)
    src = edge_src.astype(jnp.int32)
    dst = edge_dst.astype(jnp.int32)
    ea0 = edge_attr[:, :, 0].astype(jnp.float32)
    ea1 = edge_attr[:, :, 1].astype(jnp.float32)

    wqf = Wq.reshape(-1).astype(jnp.float32)
    wkf = Wk.reshape(-1).astype(jnp.float32)
    wef = We.reshape(-1).astype(jnp.float32)
    bqf = bq.reshape(-1).astype(jnp.float32)
    bkf = bk.reshape(-1).astype(jnp.float32)

    parts = [
        _sc_edge_p[p](x0, x1, src[2 * p:2 * p + 2], dst[2 * p:2 * p + 2],
                      ea0[2 * p:2 * p + 2], ea1[2 * p:2 * p + 2],
                      wqf, wkf, wef, bqf, bkf)
        for p in range(2)
    ]
    outs = []
    for p in range(2):
        tsl = slice(2 * p, 2 * p + 2)
        nsl = slice(2 * p * N_T, (2 * p + 2) * N_T)
        pqv = _finalize(parts[p], x0[nsl], x1[nsl], Wv[:, tsl], We[:, tsl],
                        Ws[:, tsl], bv[:, tsl], bs[:, tsl])
        flat = pqv.transpose(0, 1, 3, 2).reshape(2, L * N_T * 2)
        outs.append(_matvec(flat, fcW[tsl], fcb[tsl]))
    return jnp.concatenate(outs, axis=0).reshape(T, N_T, 2)


# speedup vs baseline: 1.6633x; 1.6633x over previous
import functools

import jax
import jax.numpy as jnp
from jax import lax
from jax.experimental import pallas as pl
from jax.experimental.pallas import tpu as pltpu
from jax.experimental.pallas import tpu_sc as plsc

T = 4
N_T = 1024
E = 65536
L = 2
H = 16

_N_TILES = 32
_TILES_PER_T = _N_TILES // T
_E_PER_TILE = E // _TILES_PER_T
_GROUPS = _E_PER_TILE // 16
_ACC = 10 * N_T


def _sc_edge_body(x0_hbm, x1_hbm, src_hbm, dst_hbm, ea0_hbm, ea1_hbm, wq_hbm,
                  wk_hbm, we_hbm, bq_hbm, bk_hbm, out_hbm, x0_v, x1_v, wq_v,
                  wk_v, we_v, bq_v, bk_v, acc_v, src_v, dst_v, ea0_v, ea1_v):
    cid = lax.axis_index("c")
    sid = lax.axis_index("s")
    wid = sid * 2 + cid
    t = wid // _TILES_PER_T
    s = wid % _TILES_PER_T

    pltpu.sync_copy(x0_hbm, x0_v)
    pltpu.sync_copy(x1_hbm, x1_v)
    pltpu.sync_copy(wq_hbm, wq_v)
    pltpu.sync_copy(wk_hbm, wk_v)
    pltpu.sync_copy(we_hbm, we_v)
    pltpu.sync_copy(bq_hbm, bq_v)
    pltpu.sync_copy(bk_hbm, bk_v)
    pltpu.sync_copy(src_hbm.at[t, pl.ds(s * _E_PER_TILE, _E_PER_TILE)], src_v)
    pltpu.sync_copy(dst_hbm.at[t, pl.ds(s * _E_PER_TILE, _E_PER_TILE)], dst_v)
    pltpu.sync_copy(ea0_hbm.at[t, pl.ds(s * _E_PER_TILE, _E_PER_TILE)], ea0_v)
    pltpu.sync_copy(ea1_hbm.at[t, pl.ds(s * _E_PER_TILE, _E_PER_TILE)], ea1_v)

    scale = 0.25

    def dot16(a, b):
        return jnp.sum(a * b) * scale

    cof = []
    for l in range(L):
        wq0 = wq_v[pl.ds(l * 128 + t * 32, 16)]
        wq1 = wq_v[pl.ds(l * 128 + t * 32 + 16, 16)]
        bq = bq_v[pl.ds(l * 64 + t * 16, 16)]
        bk = bk_v[pl.ds(l * 64 + t * 16, 16)]
        slots = []
        for (wsrc, rows) in ((wk_v, 2), (we_v, 2)):
            for e in range(rows):
                we_row = wsrc[pl.ds(l * 128 + t * 32 + e * 16, 16)]
                slots.append((dot16(wq0, we_row), dot16(wq1, we_row),
                              dot16(bq, we_row)))
        slots.append((dot16(wq0, bk), dot16(wq1, bk), dot16(bq, bk)))
        cof.append(slots)

    xbase = t * N_T
    zero = jnp.zeros((16,), jnp.float32)

    def z_body(i, carry):
        acc_v[pl.ds(i * 16, 16)] = zero
        return carry

    lax.fori_loop(0, _ACC // 16, z_body, 0)

    def one_group(b):
        srci = src_v[pl.ds(b, 16)]
        dsti = dst_v[pl.ds(b, 16)]
        xs0 = plsc.load_gather(x0_v, [srci])
        xs1 = plsc.load_gather(x1_v, [srci])
        di = xbase + dsti
        xd0 = plsc.load_gather(x0_v, [di])
        xd1 = plsc.load_gather(x1_v, [di])
        ea0 = ea0_v[pl.ds(b, 16)]
        ea1 = ea1_v[pl.ds(b, 16)]
        exs = []
        for l in range(L):
            u0, u1, w0, w1, s0 = [xd0 * m0 + (xd1 * m1 + c)
                                  for (m0, m1, c) in cof[l]]
            exs.append(jnp.exp((u0 * xs0 + u1 * xs1)
                               + ((w0 * ea0 + w1 * ea1) + s0)))
        for l in range(L):
            ex = exs[l]
            o = l * 5 * N_T
            plsc.addupdate_scatter(acc_v.at[pl.ds(o, N_T)], [dsti], ex)
            plsc.addupdate_scatter(acc_v.at[pl.ds(o + N_T, N_T)], [dsti],
                                   ex * xs0)
            plsc.addupdate_scatter(acc_v.at[pl.ds(o + 2 * N_T, N_T)], [dsti],
                                   ex * xs1)
            plsc.addupdate_scatter(acc_v.at[pl.ds(o + 3 * N_T, N_T)], [dsti],
                                   ex * ea0)
            plsc.addupdate_scatter(acc_v.at[pl.ds(o + 4 * N_T, N_T)], [dsti],
                                   ex * ea1)

    _UNROLL = 4

    def e_body(g, carry):
        b = g * (16 * _UNROLL)
        for u in range(_UNROLL):
            one_group(b + u * 16)
        return carry

    lax.fori_loop(0, _GROUPS // _UNROLL, e_body, 0)

    pltpu.sync_copy(acc_v, out_hbm.at[wid])


@jax.jit
def _sc_edge(x0, x1, src, dst, ea0, ea1, wq, wk, we, bq, bk):
    mesh = plsc.VectorSubcoreMesh(core_axis_name="c", subcore_axis_name="s")
    return pl.kernel(
        _sc_edge_body,
        out_type=jax.ShapeDtypeStruct((_N_TILES, _ACC), jnp.float32),
        mesh=mesh,
        scratch_types=[
            pltpu.VMEM((T * N_T,), jnp.float32),
            pltpu.VMEM((T * N_T,), jnp.float32),
            pltpu.VMEM((L * T * 32,), jnp.float32),
            pltpu.VMEM((L * T * 32,), jnp.float32),
            pltpu.VMEM((L * T * 32,), jnp.float32),
            pltpu.VMEM((L * T * 16,), jnp.float32),
            pltpu.VMEM((L * T * 16,), jnp.float32),
            pltpu.VMEM((_ACC,), jnp.float32),
            pltpu.VMEM((_E_PER_TILE,), jnp.int32),
            pltpu.VMEM((_E_PER_TILE,), jnp.int32),
            pltpu.VMEM((_E_PER_TILE,), jnp.float32),
            pltpu.VMEM((_E_PER_TILE,), jnp.float32),
        ],
        compiler_params=pltpu.CompilerParams(needs_layout_passes=False),
    )(x0, x1, src, dst, ea0, ea1, wq, wk, we, bq, bk)


def _fin_body(part_ref, x0_ref, x1_ref, wv_ref, we_ref, ws_ref, bv_ref,
              bs_ref, out_ref):
    part = part_ref[...]
    xd0 = x0_ref[...].reshape(T, N_T)
    xd1 = x1_ref[...].reshape(T, N_T)

    def tot(k):
        sl = part[:, k * N_T:(k + 1) * N_T].reshape(T, _TILES_PER_T, N_T)
        return jnp.sum(sl, axis=1)

    for l in range(L):
        o = l * 5
        den = tot(o)
        r = 1.0 / (den + 1e-16)
        mh0 = tot(o + 1) * r
        mh1 = tot(o + 2) * r
        nh0 = tot(o + 3) * r
        nh1 = tot(o + 4) * r
        sa = den * r
        for pq in range(2):
            lo, hi = pq * (H // 2), pq * (H // 2) + H // 2

            def half(ref, j):
                return jnp.sum(ref[l, :, j, lo:hi], axis=-1)[:, None]

            def halfb(ref):
                return jnp.sum(ref[l, :, lo:hi], axis=-1)[:, None]

            val = (mh0 * half(wv_ref, 0) + mh1 * half(wv_ref, 1)
                   + nh0 * half(we_ref, 0) + nh1 * half(we_ref, 1)
                   + sa * halfb(bv_ref)
                   + xd0 * half(ws_ref, 0) + xd1 * half(ws_ref, 1)
                   + halfb(bs_ref))
            out_ref[:, l, pq] = val


@jax.jit
def _finalize(part, x0, x1, Wv, We, Ws, bv, bs):
    return pl.pallas_call(
        _fin_body,
        out_shape=jax.ShapeDtypeStruct((T, L, 2, N_T), jnp.float32),
    )(part, x0, x1, Wv, We, Ws, bv, bs)


def _mv_body(f_ref, w_ref, b_ref, out_ref):
    out_ref[0] = (
        jnp.dot(f_ref[0], w_ref[0], preferred_element_type=jnp.float32)
        + b_ref[0]
    )


@jax.jit
def _matvec(flat, fcW, fcb):
    k = L * N_T * 2
    n = N_T * 2
    cb = 512
    out = pl.pallas_call(
        _mv_body,
        grid=(T, n // cb),
        in_specs=[
            pl.BlockSpec((1, 1, k), lambda t, c: (t, 0, 0)),
            pl.BlockSpec((1, k, cb), lambda t, c: (t, 0, c)),
            pl.BlockSpec((1, 1, cb), lambda t, c: (t, 0, c)),
        ],
        out_specs=pl.BlockSpec((1, 1, cb), lambda t, c: (t, 0, c)),
        out_shape=jax.ShapeDtypeStruct((T, 1, n), jnp.float32),
    )(flat.reshape(T, 1, k), fcW, fcb.reshape(T, 1, n))
    return out.reshape(T, n)


def kernel(x, edge_src, edge_dst, edge_attr, Wq, Wk, Wv, We, Ws,
           bq, bk, bv, bs, fcW, fcb):
    x0 = x[:, 0].astype(jnp.float32)
    x1 = x[:, 1].astype(jnp.float32)
    src = edge_src.astype(jnp.int32)
    dst = edge_dst.astype(jnp.int32)
    ea0 = edge_attr[:, :, 0].astype(jnp.float32)
    ea1 = edge_attr[:, :, 1].astype(jnp.float32)

    part = _sc_edge(x0, x1, src, dst, ea0, ea1,
                    Wq.reshape(-1).astype(jnp.float32),
                    Wk.reshape(-1).astype(jnp.float32),
                    We.reshape(-1).astype(jnp.float32),
                    bq.reshape(-1).astype(jnp.float32),
                    bk.reshape(-1).astype(jnp.float32))

    pqv = _finalize(part, x0, x1, Wv, We, Ws, bv, bs)
    flat = pqv.transpose(0, 1, 3, 2).reshape(T, L * N_T * 2)
    out = _matvec(flat, fcW, fcb)
    return out.reshape(T, N_T, 2)
